# baseline (speedup 1.0000x reference)
import jax
import jax.numpy as jnp
from jax import lax
from jax.experimental import pallas as pl
from jax.experimental.pallas import tpu as pltpu
from jax.experimental.pallas import tpu_sc as plsc

EMBED_DIM = 64
NB_TOKENS = 50
ROWS = 16384
VOCAB = 100001
LANES = 16
NC, NS = 2, 16
NW = NC * NS
ROWS_PER_W = ROWS // NW
BLK_ROWS = 32
NBLK = ROWS_PER_W // BLK_ROWS
DCH = EMBED_DIM // LANES
NBUF = 8

CH = 128
NCH_FULL = VOCAB // CH
REM_OFF = NCH_FULL * CH
REM = VOCAB - REM_OFF
CPW = (NCH_FULL + NW - 1) // NW
FULL_STEPS = NCH_FULL // NW
TAIL_W = NCH_FULL - FULL_STEPS * NW
CHW = CH * EMBED_DIM


def _pack_row(src, base, dst, r):
    for h in range(DCH // 2):
        a = src[pl.ds(base + h * 2 * LANES, LANES)]
        b = src[pl.ds(base + h * 2 * LANES + LANES, LANES)]
        dst[r, pl.ds(h * 2 * LANES, 2 * LANES)] = plsc.pack(
            a, b, format=plsc.PackFormat.INTERLEAVED)


def _cast_chunk(in_v, out_v):
    def row(r, carry):
        _pack_row(in_v, pl.multiple_of(r * EMBED_DIM, EMBED_DIM), out_v, r)
        return carry

    lax.fori_loop(0, CH, row, 0, unroll=8)


def _cast_kernel(tbl_hbm, out_hbm, in0_v, in1_v, out0_v, out1_v,
                 isem0, isem1, osem0, osem1):
    wid = lax.axis_index("s") * NC + lax.axis_index("c")
    inb, outb = (in0_v, in1_v), (out0_v, out1_v)
    isems, osems = (isem0, isem1), (osem0, osem1)

    def start_in(k):
        off = pl.multiple_of((wid + k * NW) * CHW, CHW)
        return pltpu.async_copy(tbl_hbm.at[pl.ds(off, CHW)],
                                inb[k % 2], isems[k % 2])

    rhandles = [None] * CPW
    whandles = [None] * CPW
    rhandles[0] = start_in(0)
    for k in range(FULL_STEPS):
        if k + 1 < FULL_STEPS:
            rhandles[k + 1] = start_in(k + 1)
        rhandles[k].wait()
        if k >= 2:
            whandles[k - 2].wait()
        _cast_chunk(inb[k % 2], outb[k % 2])
        off = pl.multiple_of((wid + k * NW) * CH, CH)
        whandles[k] = pltpu.async_copy(outb[k % 2],
                                       out_hbm.at[pl.ds(off, CH)],
                                       osems[k % 2])
    whandles[FULL_STEPS - 2].wait()
    whandles[FULL_STEPS - 1].wait()

    @pl.when(wid < TAIL_W)
    def _():
        k = FULL_STEPS
        start_in(k).wait()
        _cast_chunk(inb[k % 2], outb[k % 2])
        off = pl.multiple_of((wid + k * NW) * CH, CH)
        pltpu.sync_copy(outb[k % 2], out_hbm.at[pl.ds(off, CH)])

    @pl.when(wid == NW - 1)
    def _():
        pltpu.sync_copy(
            tbl_hbm.at[pl.ds(REM_OFF * EMBED_DIM, REM * EMBED_DIM)],
            in0_v.at[pl.ds(0, REM * EMBED_DIM)])

        def row(r, carry):
            _pack_row(in0_v, pl.multiple_of(r * EMBED_DIM, EMBED_DIM),
                      out0_v, r)
            return carry

        lax.fori_loop(0, REM, row, 0, unroll=4)
        pltpu.sync_copy(out0_v.at[pl.ds(0, REM)],
                        out_hbm.at[pl.ds(REM_OFF, REM)])


def _sc_kernel(idx_hbm, expr_hbm, table_hbm, w_hbm, b_hbm, out_hbm,
               idx_v, rows0_v, rows1_v, rows2_v, rows3_v, rows4_v, rows5_v,
               rows6_v, rows7_v, out_v, expr_v, w_v, b_v,
               sem0, sem1, sem2, sem3, sem4, sem5, sem6, sem7):
    wid = lax.axis_index("s") * NC + lax.axis_index("c")
    row0 = pl.multiple_of(wid * ROWS_PER_W, ROWS_PER_W)

    pltpu.sync_copy(expr_hbm.at[pl.ds(row0, ROWS_PER_W)], expr_v)
    pltpu.sync_copy(w_hbm, w_v)
    pltpu.sync_copy(b_hbm, b_v)

    inv_n = jnp.float32(1.0 / NB_TOKENS)
    bufs = (rows0_v, rows1_v, rows2_v, rows3_v, rows4_v, rows5_v,
            rows6_v, rows7_v)
    sems = (sem0, sem1, sem2, sem3, sem4, sem5, sem6, sem7)

    def blk_body(blk, carry):
        r0 = pl.multiple_of(row0 + blk * BLK_ROWS, BLK_ROWS)
        pltpu.sync_copy(idx_hbm.at[pl.ds(r0, BLK_ROWS)], idx_v)
        e_vecs = [expr_v[pl.ds(blk * BLK_ROWS + v * LANES, LANES)]
                  for v in range(BLK_ROWS // LANES)]

        handles = [None] * BLK_ROWS
        for p in range(NBUF - 1):
            handles[p] = pltpu.async_copy(table_hbm.at[idx_v.at[p]],
                                          bufs[p], sems[p])
        for j in range(BLK_ROWS):
            nj = j + NBUF - 1
            if nj < BLK_ROWS:
                handles[nj] = pltpu.async_copy(
                    table_hbm.at[idx_v.at[nj]], bufs[nj % NBUF],
                    sems[nj % NBUF])
            handles[j].wait()
            buf = bufs[j % NBUF]

            def tok(t, accs):
                new = []
                for h in range(DCH // 2):
                    x = buf[t, pl.ds(h * 2 * LANES, 2 * LANES)]
                    a, bb = plsc.unpack(
                        x, format=plsc.PackFormat.INTERLEAVED)
                    new.append(accs[2 * h] + a)
                    new.append(accs[2 * h + 1] + bb)
                return tuple(new)

            z = jnp.zeros((LANES,), jnp.float32)
            accs = lax.fori_loop(0, NB_TOKENS, tok, (z,) * DCH, unroll=5)
            e = e_vecs[j // LANES][j % LANES]
            for k in range(DCH):
                sl = pl.ds(k * LANES, LANES)
                out_v[j, sl] = accs[k] * inv_n + e * w_v[sl] + b_v[sl]

        pltpu.sync_copy(out_v, out_hbm.at[pl.ds(r0, BLK_ROWS)])
        return carry

    lax.fori_loop(0, NBLK, blk_body, 0)


@jax.jit
def _run(idx, expr, table, w, b):
    mesh = plsc.VectorSubcoreMesh(core_axis_name="c", subcore_axis_name="s",
                                  num_cores=NC, num_subcores=NS)
    params = pltpu.CompilerParams(use_tc_tiling_on_sc=False,
                                  needs_layout_passes=False)
    tbl_bf = pl.kernel(
        _cast_kernel,
        out_type=jax.ShapeDtypeStruct((VOCAB, EMBED_DIM), jnp.bfloat16),
        mesh=mesh,
        compiler_params=params,
        scratch_types=[
            pltpu.VMEM((CHW,), jnp.float32),
            pltpu.VMEM((CHW,), jnp.float32),
            pltpu.VMEM((CH, EMBED_DIM), jnp.bfloat16),
            pltpu.VMEM((CH, EMBED_DIM), jnp.bfloat16),
            pltpu.SemaphoreType.DMA,
            pltpu.SemaphoreType.DMA,
            pltpu.SemaphoreType.DMA,
            pltpu.SemaphoreType.DMA,
        ],
    )(table)
    return pl.kernel(
        _sc_kernel,
        out_type=jax.ShapeDtypeStruct((ROWS, EMBED_DIM), jnp.float32),
        mesh=mesh,
        compiler_params=params,
        scratch_types=[
            pltpu.VMEM((BLK_ROWS, NB_TOKENS), jnp.int32),
            pltpu.VMEM((NB_TOKENS, EMBED_DIM), jnp.bfloat16),
            pltpu.VMEM((NB_TOKENS, EMBED_DIM), jnp.bfloat16),
            pltpu.VMEM((NB_TOKENS, EMBED_DIM), jnp.bfloat16),
            pltpu.VMEM((NB_TOKENS, EMBED_DIM), jnp.bfloat16),
            pltpu.VMEM((NB_TOKENS, EMBED_DIM), jnp.bfloat16),
            pltpu.VMEM((NB_TOKENS, EMBED_DIM), jnp.bfloat16),
            pltpu.VMEM((NB_TOKENS, EMBED_DIM), jnp.bfloat16),
            pltpu.VMEM((NB_TOKENS, EMBED_DIM), jnp.bfloat16),
            pltpu.VMEM((BLK_ROWS, EMBED_DIM), jnp.float32),
            pltpu.VMEM((ROWS_PER_W,), jnp.float32),
            pltpu.VMEM((EMBED_DIM,), jnp.float32),
            pltpu.VMEM((EMBED_DIM,), jnp.float32),
            pltpu.SemaphoreType.DMA,
            pltpu.SemaphoreType.DMA,
            pltpu.SemaphoreType.DMA,
            pltpu.SemaphoreType.DMA,
            pltpu.SemaphoreType.DMA,
            pltpu.SemaphoreType.DMA,
            pltpu.SemaphoreType.DMA,
            pltpu.SemaphoreType.DMA,
        ],
    )(idx, expr, tbl_bf, w, b)


def kernel(seq_indices, expr_values, emb_table, expr_proj_w, expr_proj_b):
    w = expr_proj_w.reshape(EMBED_DIM)
    expr = expr_values.reshape(ROWS)
    tbl1d = emb_table.reshape(VOCAB * EMBED_DIM)
    return _run(seq_indices, expr, tbl1d, w, expr_proj_b)


# speedup vs baseline: 1.0307x; 1.0307x over previous
import jax
import jax.numpy as jnp
from jax import lax
from jax.experimental import pallas as pl
from jax.experimental.pallas import tpu as pltpu
from jax.experimental.pallas import tpu_sc as plsc

EMBED_DIM = 64
NB_TOKENS = 50
ROWS = 16384
VOCAB = 100001
LANES = 16
NC, NS = 2, 16
NW = NC * NS
ROWS_PER_W = ROWS // NW
BLK_ROWS = 32
NBLK = ROWS_PER_W // BLK_ROWS
DCH = EMBED_DIM // LANES
NBUF = 8

CH = 128
NCH_FULL = VOCAB // CH
REM_OFF = NCH_FULL * CH
REM = VOCAB - REM_OFF
CPW = (NCH_FULL + NW - 1) // NW
FULL_STEPS = NCH_FULL // NW
TAIL_W = NCH_FULL - FULL_STEPS * NW
CHW = CH * EMBED_DIM


def _pack_row(src, base, dst, r):
    for h in range(DCH // 2):
        a = src[pl.ds(base + h * 2 * LANES, LANES)]
        b = src[pl.ds(base + h * 2 * LANES + LANES, LANES)]
        dst[r, pl.ds(h * 2 * LANES, 2 * LANES)] = plsc.pack(
            a, b, format=plsc.PackFormat.INTERLEAVED)


def _cast_chunk(in_v, out_v):
    def row(r, carry):
        _pack_row(in_v, pl.multiple_of(r * EMBED_DIM, EMBED_DIM), out_v, r)
        return carry

    lax.fori_loop(0, CH, row, 0, unroll=4)


def _cast_kernel(tbl_hbm, out_hbm, in0_v, in1_v, out0_v, out1_v,
                 isem0, isem1, osem0, osem1):
    wid = lax.axis_index("s") * NC + lax.axis_index("c")
    inb, outb = (in0_v, in1_v), (out0_v, out1_v)
    isems, osems = (isem0, isem1), (osem0, osem1)

    def start_in(k):
        off = pl.multiple_of((wid + k * NW) * CHW, CHW)
        return pltpu.async_copy(tbl_hbm.at[pl.ds(off, CHW)],
                                inb[k % 2], isems[k % 2])

    rhandles = [None] * CPW
    whandles = [None] * CPW
    rhandles[0] = start_in(0)
    for k in range(FULL_STEPS):
        if k + 1 < FULL_STEPS:
            rhandles[k + 1] = start_in(k + 1)
        rhandles[k].wait()
        if k >= 2:
            whandles[k - 2].wait()
        _cast_chunk(inb[k % 2], outb[k % 2])
        off = pl.multiple_of((wid + k * NW) * CH, CH)
        whandles[k] = pltpu.async_copy(outb[k % 2],
                                       out_hbm.at[pl.ds(off, CH)],
                                       osems[k % 2])
    whandles[FULL_STEPS - 2].wait()
    whandles[FULL_STEPS - 1].wait()

    @pl.when(wid < TAIL_W)
    def _():
        k = FULL_STEPS
        start_in(k).wait()
        _cast_chunk(inb[k % 2], outb[k % 2])
        off = pl.multiple_of((wid + k * NW) * CH, CH)
        pltpu.sync_copy(outb[k % 2], out_hbm.at[pl.ds(off, CH)])

    @pl.when(wid == NW - 1)
    def _():
        pltpu.sync_copy(
            tbl_hbm.at[pl.ds(REM_OFF * EMBED_DIM, REM * EMBED_DIM)],
            in0_v.at[pl.ds(0, REM * EMBED_DIM)])

        def row(r, carry):
            _pack_row(in0_v, pl.multiple_of(r * EMBED_DIM, EMBED_DIM),
                      out0_v, r)
            return carry

        lax.fori_loop(0, REM, row, 0, unroll=4)
        pltpu.sync_copy(out0_v.at[pl.ds(0, REM)],
                        out_hbm.at[pl.ds(REM_OFF, REM)])


def _sc_kernel(idx_hbm, expr_hbm, table_hbm, w_hbm, b_hbm, out_hbm,
               idx_v, idx2_v, rows0_v, rows1_v, rows2_v, rows3_v,
               rows4_v, rows5_v, rows6_v, rows7_v, out_v, expr_v, w_v, b_v,
               sem0, sem1, sem2, sem3, sem4, sem5, sem6, sem7,
               isem_a, isem_b):
    wid = lax.axis_index("s") * NC + lax.axis_index("c")
    row0 = pl.multiple_of(wid * ROWS_PER_W, ROWS_PER_W)

    pltpu.sync_copy(expr_hbm.at[pl.ds(row0, ROWS_PER_W)], expr_v)
    pltpu.sync_copy(w_hbm, w_v)
    pltpu.sync_copy(b_hbm, b_v)

    inv_n = jnp.float32(1.0 / NB_TOKENS)
    bufs = (rows0_v, rows1_v, rows2_v, rows3_v, rows4_v, rows5_v,
            rows6_v, rows7_v)
    sems = (sem0, sem1, sem2, sem3, sem4, sem5, sem6, sem7)

    def start_idx(blk, ibuf, isem):
        r0 = pl.multiple_of(row0 + (blk % NBLK) * BLK_ROWS, BLK_ROWS)
        return pltpu.async_copy(idx_hbm.at[pl.ds(r0, BLK_ROWS)], ibuf, isem)

    def do_blk(blk, ibuf):
        e_vecs = [expr_v[pl.ds(blk * BLK_ROWS + v * LANES, LANES)]
                  for v in range(BLK_ROWS // LANES)]

        handles = [None] * BLK_ROWS
        for p in range(NBUF - 1):
            handles[p] = pltpu.async_copy(table_hbm.at[ibuf.at[p]],
                                          bufs[p], sems[p])
        for j in range(BLK_ROWS):
            nj = j + NBUF - 1
            if nj < BLK_ROWS:
                handles[nj] = pltpu.async_copy(
                    table_hbm.at[ibuf.at[nj]], bufs[nj % NBUF],
                    sems[nj % NBUF])
            handles[j].wait()
            buf = bufs[j % NBUF]

            def tok(t, accs):
                new = []
                for h in range(DCH // 2):
                    x = buf[t, pl.ds(h * 2 * LANES, 2 * LANES)]
                    a, bb = plsc.unpack(
                        x, format=plsc.PackFormat.INTERLEAVED)
                    new.append(accs[2 * h] + a)
                    new.append(accs[2 * h + 1] + bb)
                return tuple(new)

            z = jnp.zeros((LANES,), jnp.float32)
            accs = lax.fori_loop(0, NB_TOKENS, tok, (z,) * DCH, unroll=5)
            e = e_vecs[j // LANES][j % LANES]
            for k in range(DCH):
                sl = pl.ds(k * LANES, LANES)
                out_v[j, sl] = accs[k] * inv_n + e * w_v[sl] + b_v[sl]

        r0 = pl.multiple_of(row0 + blk * BLK_ROWS, BLK_ROWS)
        pltpu.sync_copy(out_v, out_hbm.at[pl.ds(r0, BLK_ROWS)])

    h_idx0 = start_idx(0, idx_v, isem_a)

    def super_body(sb, carry):
        blk0 = sb * 2
        h0 = pltpu.make_async_copy(idx_hbm.at[pl.ds(row0, BLK_ROWS)],
                                   idx_v, isem_a)
        h0.wait()
        start_idx(blk0 + 1, idx2_v, isem_b)
        do_blk(blk0, idx_v)
        h1 = pltpu.make_async_copy(idx_hbm.at[pl.ds(row0, BLK_ROWS)],
                                   idx2_v, isem_b)
        h1.wait()
        start_idx(blk0 + 2, idx_v, isem_a)
        do_blk(blk0 + 1, idx2_v)
        return carry

    lax.fori_loop(0, NBLK // 2, super_body, 0)
    pltpu.make_async_copy(idx_hbm.at[pl.ds(row0, BLK_ROWS)],
                          idx_v, isem_a).wait()


@jax.jit
def _run(idx, expr, table, w, b):
    mesh = plsc.VectorSubcoreMesh(core_axis_name="c", subcore_axis_name="s",
                                  num_cores=NC, num_subcores=NS)
    params = pltpu.CompilerParams(use_tc_tiling_on_sc=False,
                                  needs_layout_passes=False)
    tbl_bf = pl.kernel(
        _cast_kernel,
        out_type=jax.ShapeDtypeStruct((VOCAB, EMBED_DIM), jnp.bfloat16),
        mesh=mesh,
        compiler_params=params,
        scratch_types=[
            pltpu.VMEM((CHW,), jnp.float32),
            pltpu.VMEM((CHW,), jnp.float32),
            pltpu.VMEM((CH, EMBED_DIM), jnp.bfloat16),
            pltpu.VMEM((CH, EMBED_DIM), jnp.bfloat16),
            pltpu.SemaphoreType.DMA,
            pltpu.SemaphoreType.DMA,
            pltpu.SemaphoreType.DMA,
            pltpu.SemaphoreType.DMA,
        ],
    )(table)
    return pl.kernel(
        _sc_kernel,
        out_type=jax.ShapeDtypeStruct((ROWS, EMBED_DIM), jnp.float32),
        mesh=mesh,
        compiler_params=params,
        scratch_types=[
            pltpu.VMEM((BLK_ROWS, NB_TOKENS), jnp.int32),
            pltpu.VMEM((BLK_ROWS, NB_TOKENS), jnp.int32),
            pltpu.VMEM((NB_TOKENS, EMBED_DIM), jnp.bfloat16),
            pltpu.VMEM((NB_TOKENS, EMBED_DIM), jnp.bfloat16),
            pltpu.VMEM((NB_TOKENS, EMBED_DIM), jnp.bfloat16),
            pltpu.VMEM((NB_TOKENS, EMBED_DIM), jnp.bfloat16),
            pltpu.VMEM((NB_TOKENS, EMBED_DIM), jnp.bfloat16),
            pltpu.VMEM((NB_TOKENS, EMBED_DIM), jnp.bfloat16),
            pltpu.VMEM((NB_TOKENS, EMBED_DIM), jnp.bfloat16),
            pltpu.VMEM((NB_TOKENS, EMBED_DIM), jnp.bfloat16),
            pltpu.VMEM((BLK_ROWS, EMBED_DIM), jnp.float32),
            pltpu.VMEM((ROWS_PER_W,), jnp.float32),
            pltpu.VMEM((EMBED_DIM,), jnp.float32),
            pltpu.VMEM((EMBED_DIM,), jnp.float32),
            pltpu.SemaphoreType.DMA,
            pltpu.SemaphoreType.DMA,
            pltpu.SemaphoreType.DMA,
            pltpu.SemaphoreType.DMA,
            pltpu.SemaphoreType.DMA,
            pltpu.SemaphoreType.DMA,
            pltpu.SemaphoreType.DMA,
            pltpu.SemaphoreType.DMA,
            pltpu.SemaphoreType.DMA,
            pltpu.SemaphoreType.DMA,
        ],
    )(idx, expr, tbl_bf, w, b)


def kernel(seq_indices, expr_values, emb_table, expr_proj_w, expr_proj_b):
    w = expr_proj_w.reshape(EMBED_DIM)
    expr = expr_values.reshape(ROWS)
    tbl1d = emb_table.reshape(VOCAB * EMBED_DIM)
    return _run(seq_indices, expr, tbl1d, w, expr_proj_b)
